# Initial kernel scaffold; baseline (speedup 1.0000x reference)
import jax
import jax.numpy as jnp
from jax.experimental import pallas as pl


def kernel(uv, feat_2d, feat_3d, W1, b1, W2, b2, W3, b3):
    raise NotImplementedError("write your pallas kernel here")


# speedup vs baseline: 61.7084x; 61.7084x over previous
import functools

import jax
import jax.numpy as jnp
from jax.experimental import pallas as pl
from jax.experimental.pallas import tpu as pltpu

_BS, _H, _W, _N, _C, _K = 2, 60, 80, 4096, 64, 3
_M = _H * _W
_MT = 256
_MPAD = 4864
_GROWS = 72


def _body(p_ref, pb_ref, g_ref, w1_ref, b1_ref, w2_ref, b2_ref, w3_ref,
          b3_ref, out_ref):
    t = pl.program_id(1)

    m = jax.lax.broadcasted_iota(jnp.int32, (1, _MT), 1) + t * _MT
    qx = (m % _W).astype(jnp.float32)
    qy = (m // _W).astype(jnp.float32)
    qq = qx * qx + qy * qy

    pxb = pb_ref[0, :, 0:1].astype(jnp.float32)
    pyb = pb_ref[0, :, 1:2].astype(jnp.float32)
    pp = p_ref[0, :, 0:1]

    qp = qx * pxb + qy * pyb
    d2 = (qq - 2.0 * qp) + pp

    niota = jax.lax.broadcasted_iota(jnp.int32, (_N, _MT), 0)
    gmat = g_ref[0]

    final = jnp.zeros((_C, _MT), dtype=jnp.float32)
    for k in range(_K):
        minv = jnp.min(d2, axis=0, keepdims=True)
        eq = d2 == minv
        idx = jnp.min(jnp.where(eq, niota, _N), axis=0, keepdims=True)
        sel = niota == idx
        if k + 1 < _K:
            d2 = jnp.where(sel, jnp.float32(3.0e38), d2)

        onehot = jnp.where(sel, jnp.float32(1), jnp.float32(0)).astype(jnp.bfloat16)
        g = jax.lax.dot_general(gmat, onehot, (((1,), (0,)), ((), ())),
                                preferred_element_type=jnp.float32)
        ox = (g[0:1] + g[1:2]) - qx
        oy = (g[2:3] + g[3:4]) - qy
        norm = jnp.sqrt(ox * ox + oy * oy)

        h1 = (w1_ref[:, 0:1] * ox + w1_ref[:, 1:2] * oy
              + w1_ref[:, 2:3] * norm + b1_ref[...])
        h1 = jnp.where(h1 >= 0, h1, 0.1 * h1)
        s = jax.lax.dot_general(w2_ref[...].astype(jnp.bfloat16),
                                h1.astype(jnp.bfloat16),
                                (((1,), (0,)), ((), ())),
                                preferred_element_type=jnp.float32)
        s = jax.nn.sigmoid(s + b2_ref[...])
        final = final + s * g[4:4 + _C]

    out = jax.lax.dot_general(w3_ref[...].astype(jnp.bfloat16),
                              final.astype(jnp.bfloat16),
                              (((1,), (0,)), ((), ())),
                              preferred_element_type=jnp.float32)
    out = out + b3_ref[...]
    out_ref[0] = jnp.where(out >= 0, out, 0.1 * out)


@jax.jit
def kernel(uv, feat_2d, feat_3d, W1, b1, W2, b2, W3, b3):
    bs = uv.shape[0]
    del feat_2d

    p = jnp.swapaxes(uv, 1, 2)
    pp = jnp.sum(p * p, axis=-1)[..., None]
    pb = p.astype(jnp.bfloat16)

    uv_hi = uv.astype(jnp.bfloat16)
    uv_lo = (uv - uv_hi.astype(jnp.float32)).astype(jnp.bfloat16)
    gmat = jnp.concatenate(
        [uv_hi[:, 0:1], uv_lo[:, 0:1], uv_hi[:, 1:2], uv_lo[:, 1:2],
         feat_3d.astype(jnp.bfloat16),
         jnp.zeros((bs, _GROWS - 4 - _C, _N), jnp.bfloat16)], axis=1)

    out = pl.pallas_call(
        _body,
        grid=(bs, _MPAD // _MT),
        in_specs=[
            pl.BlockSpec((1, _N, 1), lambda b, t: (b, 0, 0)),
            pl.BlockSpec((1, _N, 2), lambda b, t: (b, 0, 0)),
            pl.BlockSpec((1, _GROWS, _N), lambda b, t: (b, 0, 0)),
            pl.BlockSpec((16, 3), lambda b, t: (0, 0)),
            pl.BlockSpec((16, 1), lambda b, t: (0, 0)),
            pl.BlockSpec((_C, 16), lambda b, t: (0, 0)),
            pl.BlockSpec((_C, 1), lambda b, t: (0, 0)),
            pl.BlockSpec((_C, _C), lambda b, t: (0, 0)),
            pl.BlockSpec((_C, 1), lambda b, t: (0, 0)),
        ],
        out_specs=pl.BlockSpec((1, _C, _MT), lambda b, t: (b, 0, t)),
        out_shape=jax.ShapeDtypeStruct((bs, _C, _MPAD), jnp.float32),
    )(pp, pb, gmat, W1, b1.reshape(-1, 1), W2, b2.reshape(-1, 1), W3,
      b3.reshape(-1, 1))

    return out[:, :, :_M].reshape(bs, _C, _H, _W)
